# baseline (speedup 1.0000x reference)
import jax
import jax.numpy as jnp
from jax.experimental import pallas as pl
from jax.experimental.pallas import tpu as pltpu

_B, _S, _C = 4096, 50, 256
_N = _B * _S
_R = 2048


def _onehot_block(x_ref, out_ref):
    idx = x_ref[...]
    iota = jax.lax.broadcasted_iota(jnp.int32, (_R, _C), 1)
    out_ref[...] = (idx == iota).astype(jnp.float32)


def kernel(x):
    xf = x.reshape(_N, 1).astype(jnp.int32)
    out = pl.pallas_call(
        _onehot_block,
        grid=(_N // _R,),
        in_specs=[pl.BlockSpec((_R, 1), lambda i: (i, 0))],
        out_specs=pl.BlockSpec((_R, _C), lambda i: (i, 0)),
        out_shape=jax.ShapeDtypeStruct((_N, _C), jnp.float32),
        compiler_params=pltpu.CompilerParams(
            dimension_semantics=("parallel",),
        ),
    )(xf)
    return out.reshape(_B, _S, _C)


# speedup vs baseline: 1.2071x; 1.2071x over previous
import jax
import jax.numpy as jnp
from jax.experimental import pallas as pl
from jax.experimental.pallas import tpu as pltpu

_B, _S, _C = 4096, 50, 256
_N = _B * _S
_R = 2048


def _onehot_block(x_ref, out_ref):
    idx = x_ref[0].reshape(_R, 1)
    iota = jax.lax.broadcasted_iota(jnp.int32, (_R, _C), 1)
    out_ref[...] = (idx == iota).astype(jnp.float32)


def kernel(x):
    xf = x.reshape(_N // _R, 1, _R).astype(jnp.int32)
    out = pl.pallas_call(
        _onehot_block,
        grid=(_N // _R,),
        in_specs=[pl.BlockSpec((1, 1, _R), lambda i: (i, 0, 0))],
        out_specs=pl.BlockSpec((_R, _C), lambda i: (i, 0)),
        out_shape=jax.ShapeDtypeStruct((_N, _C), jnp.float32),
        compiler_params=pltpu.CompilerParams(
            dimension_semantics=("parallel",),
        ),
    )(xf)
    return out.reshape(_B, _S, _C)


# speedup vs baseline: 2.4779x; 2.0528x over previous
import jax
import jax.numpy as jnp
from jax.experimental import pallas as pl
from jax.experimental.pallas import tpu as pltpu

_B, _S, _C = 4096, 50, 256
_RB = 64


def _onehot_block(x_ref, out_ref):
    idx = x_ref[...]
    iota = jax.lax.broadcasted_iota(jnp.int32, (_RB, _S, _C), 2)
    out_ref[...] = (idx[:, :, None] == iota).astype(jnp.float32)


def kernel(x):
    return pl.pallas_call(
        _onehot_block,
        grid=(_B // _RB,),
        in_specs=[pl.BlockSpec((_RB, _S), lambda i: (i, 0))],
        out_specs=pl.BlockSpec((_RB, _S, _C), lambda i: (i, 0, 0)),
        out_shape=jax.ShapeDtypeStruct((_B, _S, _C), jnp.float32),
        compiler_params=pltpu.CompilerParams(
            dimension_semantics=("parallel",),
        ),
    )(x.astype(jnp.int32))


# speedup vs baseline: 2.5566x; 1.0317x over previous
import jax
import jax.numpy as jnp
from jax.experimental import pallas as pl
from jax.experimental.pallas import tpu as pltpu

_B, _S, _C = 4096, 50, 256
_RB = 128


def _onehot_block(x_ref, out_ref):
    idx = x_ref[...]
    iota = jax.lax.broadcasted_iota(jnp.int32, (_RB, _S, _C), 2)
    out_ref[...] = (idx[:, :, None] == iota).astype(jnp.float32)


def kernel(x):
    return pl.pallas_call(
        _onehot_block,
        grid=(_B // _RB,),
        in_specs=[pl.BlockSpec((_RB, _S), lambda i: (i, 0))],
        out_specs=pl.BlockSpec((_RB, _S, _C), lambda i: (i, 0, 0)),
        out_shape=jax.ShapeDtypeStruct((_B, _S, _C), jnp.float32),
        compiler_params=pltpu.CompilerParams(
            dimension_semantics=("parallel",),
        ),
    )(x.astype(jnp.int32))


# speedup vs baseline: 2.5592x; 1.0010x over previous
import jax
import jax.numpy as jnp
from jax.experimental import pallas as pl
from jax.experimental.pallas import tpu as pltpu

_B, _S, _C = 4096, 50, 256
_RB = 64
_G = _B // _RB
_NBUF = 4


def _onehot_block(x_ref, out_ref, sbuf, sems):
    i = pl.program_id(0)
    slot = jax.lax.rem(i, _NBUF)

    def copy(j, s):
        return pltpu.make_async_copy(
            sbuf.at[s], out_ref.at[pl.ds(j * _RB, _RB), :, :], sems.at[s]
        )

    @pl.when(i >= _NBUF)
    def _wait_prev():
        copy(i - _NBUF, slot).wait()

    idx = x_ref[...]
    iota = jax.lax.broadcasted_iota(jnp.int32, (_RB, _S, _C), 2)
    sbuf[slot] = (idx[:, :, None] == iota).astype(jnp.float32)
    copy(i, slot).start()

    @pl.when(i == _G - 1)
    def _drain():
        for d in range(_NBUF):
            j = _G - _NBUF + d
            copy(j, jax.lax.rem(j, _NBUF)).wait()


def kernel(x):
    return pl.pallas_call(
        _onehot_block,
        grid=(_G,),
        in_specs=[pl.BlockSpec((_RB, _S), lambda i: (i, 0))],
        out_specs=pl.BlockSpec(memory_space=pltpu.MemorySpace.HBM),
        out_shape=jax.ShapeDtypeStruct((_B, _S, _C), jnp.float32),
        scratch_shapes=[
            pltpu.VMEM((_NBUF, _RB, _S, _C), jnp.float32),
            pltpu.SemaphoreType.DMA((_NBUF,)),
        ],
        compiler_params=pltpu.CompilerParams(
            dimension_semantics=("arbitrary",),
        ),
    )(x.astype(jnp.int32))
